# baseline (speedup 1.0000x reference)
import jax
import jax.numpy as jnp
from jax import lax
from jax.experimental import pallas as pl
from jax.experimental.pallas import tpu as pltpu
from jax.experimental.pallas import tpu_sc as plsc

EMBED = 64
BATCH = 16384
NC = 2
NS = 16
L = 16
NW = NC * NS
BPW = BATCH // NW
C = 32
NCH = BPW // C


def _gather_u_body(u_hbm, ut_hbm, stage_hbm, uidx, ubuf, ubuf2, drain,
                   sem, sem2, osem):
    wid = lax.axis_index("s") * NC + lax.axis_index("c")
    base = wid * BPW
    pltpu.sync_copy(u_hbm.at[wid], uidx)
    ubufs = (ubuf, ubuf2)
    sems = (sem, sem2)

    def fire(ci, b):
        for g in range(C // L):
            uvec = uidx[pl.ds(ci * C + g * L, L)]
            for r in range(L):
                pltpu.async_copy(ut_hbm.at[uvec[r]], ubufs[b].at[g * L + r],
                                 sems[b])

    for b in range(2):
        fire(b, b)

    def pairs(p, carry):
        for b in range(2):
            ci = 2 * p + b
            pltpu.make_async_copy(
                ut_hbm.at[pl.ds(0, C)], drain, sems[b]).wait()
            pltpu.async_copy(
                ubufs[b], stage_hbm.at[pl.ds(base + ci * C, C)], osem)
            pltpu.make_async_copy(
                ut_hbm.at[pl.ds(0, C)], drain, osem).wait()

            @pl.when(ci < NCH - 2)
            def _():
                fire(ci + 2, b)
        return carry

    lax.fori_loop(0, NCH // 2, pairs, 0)


def _dot_v_body(v_hbm, it_hbm, stage_hbm, out_hbm,
                vidx, ubuf, ubuf2, vbuf, vbuf2, drain, outv, sem, sem2):
    wid = lax.axis_index("s") * NC + lax.axis_index("c")
    base = wid * BPW
    pltpu.sync_copy(v_hbm.at[wid], vidx)
    lanes = lax.iota(jnp.int32, L)
    ubufs = (ubuf, ubuf2)
    vbufs = (vbuf, vbuf2)
    sems = (sem, sem2)

    def fire(ci, b):
        pltpu.async_copy(
            stage_hbm.at[pl.ds(base + ci * C, C)], ubufs[b], sems[b])
        for g in range(C // L):
            vvec = vidx[pl.ds(ci * C + g * L, L)]
            for r in range(L):
                pltpu.async_copy(it_hbm.at[vvec[r]], vbufs[b].at[g * L + r],
                                 sems[b])

    for b in range(2):
        fire(b, b)

    def pairs(p, carry):
        for b in range(2):
            ci = 2 * p + b
            pltpu.make_async_copy(
                it_hbm.at[pl.ds(0, C)], drain, sems[b]).wait()
            pltpu.make_async_copy(
                it_hbm.at[pl.ds(0, C)], drain, sems[b]).wait()
            for g in range(C // L):
                tot = jnp.zeros((L,), jnp.float32)
                for r in range(L):
                    j = g * L + r
                    acc = ubufs[b][j, pl.ds(0, L)] * vbufs[b][j, pl.ds(0, L)]
                    for e in range(1, EMBED // L):
                        acc = acc + (ubufs[b][j, pl.ds(e * L, L)]
                                     * vbufs[b][j, pl.ds(e * L, L)])
                    tot = jnp.where(lanes == r, jnp.sum(acc), tot)
                outv[pl.ds(ci * C + g * L, L)] = tot

            @pl.when(ci < NCH - 2)
            def _():
                fire(ci + 2, b)
        return carry

    lax.fori_loop(0, NCH // 2, pairs, 0)
    pltpu.sync_copy(outv, out_hbm.at[pl.ds(base, BPW)])


def kernel(u, v, user_table, item_table):
    u2 = u.astype(jnp.int32).reshape(NW, BPW)
    v2 = v.astype(jnp.int32).reshape(NW, BPW)
    mesh = plsc.VectorSubcoreMesh(core_axis_name="c", subcore_axis_name="s")
    params = pltpu.CompilerParams(
        needs_layout_passes=False, use_tc_tiling_on_sc=True)

    gather_u = pl.kernel(
        _gather_u_body,
        out_type=jax.ShapeDtypeStruct((BATCH, EMBED), jnp.float32),
        mesh=mesh,
        compiler_params=params,
        scratch_types=[
            pltpu.VMEM((BPW,), jnp.int32),
            pltpu.VMEM((C, EMBED), jnp.float32),
            pltpu.VMEM((C, EMBED), jnp.float32),
            pltpu.VMEM((C, EMBED), jnp.float32),
            pltpu.SemaphoreType.DMA,
            pltpu.SemaphoreType.DMA,
            pltpu.SemaphoreType.DMA,
        ],
    )
    stage = gather_u(u2, user_table)

    dot_v = pl.kernel(
        _dot_v_body,
        out_type=jax.ShapeDtypeStruct((BATCH,), jnp.float32),
        mesh=mesh,
        compiler_params=params,
        scratch_types=[
            pltpu.VMEM((BPW,), jnp.int32),
            pltpu.VMEM((C, EMBED), jnp.float32),
            pltpu.VMEM((C, EMBED), jnp.float32),
            pltpu.VMEM((C, EMBED), jnp.float32),
            pltpu.VMEM((C, EMBED), jnp.float32),
            pltpu.VMEM((C, EMBED), jnp.float32),
            pltpu.VMEM((BPW,), jnp.float32),
            pltpu.SemaphoreType.DMA,
            pltpu.SemaphoreType.DMA,
        ],
    )
    return dot_v(v2, item_table, stage)
